# baseline (speedup 1.0000x reference)
import functools

import jax
import jax.numpy as jnp
from jax import lax
from jax.experimental import pallas as pl
from jax.experimental.pallas import tpu as pltpu
from jax.experimental.pallas import tpu_sc as plsc

_NUM_FIELDS = 26
_VOCAB = 100000
_DIM = 32
_BATCH = 16384

_NC = 2
_NS = 16
_L = 16

_NW = _NC * _NS
_BPT = _BATCH // _NW
_QTR = _BPT // 4
_SROWS = _VOCAB * _NUM_FIELDS * _DIM // 128
_ORDIM = _NUM_FIELDS * _DIM


def _body(idx_hbm, table_hbm, fixed_hbm, out_hbm,
          idx_v, gidx, qarr, gbuf0, gbuf1, tbuf0, tbuf1,
          part_v, tot_v, fixed_v, sums_v, shared,
          gs0, gs1, ws0, ws1):
    c = lax.axis_index("c")
    s = lax.axis_index("s")
    wid = s * _NC + c
    b0 = wid * _BPT
    ob0 = (s * _NC + (1 - c)) * _BPT

    pltpu.sync_copy(fixed_hbm, fixed_v)

    pltpu.sync_copy(idx_hbm.at[pl.ds(0, _NUM_FIELDS), pl.ds(ob0, _BPT)], idx_v)

    def sum_field_a(f, carry):
        def add_vec(j, acc):
            return acc + idx_v[f, pl.ds(j * _L, _L)]

        part_v[pl.ds(f * _L, _L)] = lax.fori_loop(
            0, _BPT // _L, add_vec, jnp.zeros((_L,), jnp.int32))
        return carry

    lax.fori_loop(0, _NUM_FIELDS, sum_field_a, 0)

    pltpu.sync_copy(idx_hbm.at[pl.ds(0, _NUM_FIELDS), pl.ds(b0, _BPT)], idx_v)

    def sum_field_b(f, carry):
        fbase = f * _BPT

        def step(j, acc):
            v = idx_v[f, pl.ds(j * _L, _L)]
            flat = v + f * _VOCAB
            gidx[pl.ds(fbase + j * _L, _L)] = lax.shift_right_logical(flat, 2)
            qarr[pl.ds(fbase + j * _L, _L)] = lax.bitwise_and(flat, 3)
            return acc + v

        acc0 = part_v[pl.ds(f * _L, _L)]
        part_v[pl.ds(f * _L, _L)] = lax.fori_loop(0, _BPT // _L, step, acc0)
        return carry

    lax.fori_loop(0, _NUM_FIELDS, sum_field_b, 0)

    pltpu.sync_copy(part_v, shared.at[s])
    plsc.subcore_barrier()
    pltpu.sync_copy(shared, sums_v)

    def tot_field(f, carry):
        def add_sub(r, acc):
            return acc + sums_v[r, pl.ds(f * _L, _L)]

        t = lax.fori_loop(0, _NS, add_sub, jnp.zeros((_L,), jnp.int32))
        tot_v[pl.ds(f * _L, _L)] = t
        return carry + jnp.where(jnp.sum(t) == 0, 1, 0)

    n_masked = lax.fori_loop(0, _NUM_FIELDS, tot_field, 0)
    any_masked = n_masked > 0

    lanes = lax.iota(jnp.int32, _L)

    def transpose_stage(st, gbuf, tbuf):
        f = lax.div(st, 4)
        gbase = f * _BPT + lax.rem(st, 4) * _QTR

        @pl.when(jnp.logical_not(any_masked) |
                 (jnp.sum(tot_v[pl.ds(f * _L, _L)]) != 0))
        def _():
            def grp(gi, carry):
                gb = gi * _L
                rows = gb + lanes
                qv32 = qarr[pl.ds(gbase + gb, _L)] * _DIM
                for d in range(_DIM):
                    vals = plsc.load_gather(gbuf, [rows, qv32 + d])
                    tbuf[d, pl.ds(gb, _L)] = vals
                return carry

            lax.fori_loop(0, _QTR // _L, grp, 0)

        @pl.when(any_masked & (jnp.sum(tot_v[pl.ds(f * _L, _L)]) == 0))
        def _():
            for d in range(_DIM):
                fv = fixed_v[pl.ds((d // _L) * _L, _L)][d % _L]
                splat = jnp.full((_L,), fv, jnp.float32)

                def fill(gi, carry):
                    tbuf[d, pl.ds(gi * _L, _L)] = splat
                    return carry

                lax.fori_loop(0, _QTR // _L, fill, 0)

    def fire_gather(st, gbuf, sem):
        gbase = lax.div(st, 4) * _BPT + lax.rem(st, 4) * _QTR
        pltpu.make_async_copy(
            table_hbm.at[gidx.at[pl.ds(gbase, _QTR)]], gbuf, sem).start()

    def wait_gather(st, gbuf, sem):
        gbase = lax.div(st, 4) * _BPT + lax.rem(st, 4) * _QTR
        pltpu.make_async_copy(
            table_hbm.at[gidx.at[pl.ds(gbase, _QTR)]], gbuf, sem).wait()

    def out_slab(st):
        f = lax.div(st, 4)
        col = b0 + lax.rem(st, 4) * _QTR
        return out_hbm.at[pl.ds(f * _DIM, _DIM), pl.ds(col, _QTR)]

    def pair(g, carry):
        st0 = 2 * g
        st1 = st0 + 1

        @pl.when(g > 0)
        def _():
            pltpu.make_async_copy(tbuf0, out_slab(st0), ws0).wait()

        fire_gather(st0, gbuf0, gs0)

        @pl.when(g > 0)
        def _():
            pltpu.make_async_copy(tbuf1, out_slab(st1), ws1).wait()

        fire_gather(st1, gbuf1, gs1)

        wait_gather(st0, gbuf0, gs0)
        transpose_stage(st0, gbuf0, tbuf0)
        pltpu.make_async_copy(tbuf0, out_slab(st0), ws0).start()

        wait_gather(st1, gbuf1, gs1)
        transpose_stage(st1, gbuf1, tbuf1)
        pltpu.make_async_copy(tbuf1, out_slab(st1), ws1).start()
        return carry

    lax.fori_loop(0, _NUM_FIELDS * 2, pair, 0)
    pltpu.make_async_copy(tbuf0, out_slab(0), ws0).wait()
    pltpu.make_async_copy(tbuf1, out_slab(1), ws1).wait()


@functools.partial(
    pl.kernel,
    out_type=jax.ShapeDtypeStruct((_ORDIM, _BATCH), jnp.float32),
    mesh=plsc.VectorSubcoreMesh(core_axis_name="c", subcore_axis_name="s"),
    compiler_params=pltpu.CompilerParams(
        needs_layout_passes=False, use_tc_tiling_on_sc=True),
    scratch_types=[
        pltpu.VMEM((_NUM_FIELDS, _BPT), jnp.int32),
        pltpu.VMEM((_NUM_FIELDS * _BPT,), jnp.int32),
        pltpu.VMEM((_NUM_FIELDS * _BPT,), jnp.int32),
        pltpu.VMEM((_QTR, 128), jnp.float32),
        pltpu.VMEM((_QTR, 128), jnp.float32),
        pltpu.VMEM((_DIM, _QTR), jnp.float32),
        pltpu.VMEM((_DIM, _QTR), jnp.float32),
        pltpu.VMEM((_NUM_FIELDS * _L,), jnp.int32),
        pltpu.VMEM((_NUM_FIELDS * _L,), jnp.int32),
        pltpu.VMEM((_DIM,), jnp.float32),
        pltpu.VMEM((_NS, _NUM_FIELDS * _L), jnp.int32),
        pltpu.VMEM_SHARED((_NS, _NUM_FIELDS * _L), jnp.int32),
        pltpu.SemaphoreType.DMA,
        pltpu.SemaphoreType.DMA,
        pltpu.SemaphoreType.DMA,
        pltpu.SemaphoreType.DMA,
    ],
)
def _sc_embedding(idx_hbm, table_hbm, fixed_hbm, out_hbm, *scratch):
    _body(idx_hbm, table_hbm, fixed_hbm, out_hbm, *scratch)


def kernel(sparse_inputs, tables, fixed_vector):
    idx_t = sparse_inputs.astype(jnp.int32).T
    table128 = tables.reshape(_SROWS, 128)
    fixed = fixed_vector.astype(jnp.float32)
    out2 = _sc_embedding(idx_t, table128, fixed)
    return out2.reshape(_NUM_FIELDS, _DIM, _BATCH).transpose(2, 0, 1)


# speedup vs baseline: 4.3016x; 4.3016x over previous
import functools

import jax
import jax.numpy as jnp
from jax import lax
from jax.experimental import pallas as pl
from jax.experimental.pallas import tpu as pltpu
from jax.experimental.pallas import tpu_sc as plsc

_NUM_FIELDS = 26
_VOCAB = 100000
_DIM = 32
_BATCH = 16384

_NC = 2
_NS = 16
_L = 16

_NW = _NC * _NS
_BPT = _BATCH // _NW
_HSLAB = _BPT // 2
_NROWS = _NUM_FIELDS * _DIM
_RPT = _NROWS // _NW
_QB = 4096
_NQ = _BATCH // _QB


def _body(idx_hbm, table_hbm, fixed_hbm, out_hbm,
          rbuf, ibuf, obuf, slab_v, part_v, tot_v, fixed_v, sums4, shared):
    c = lax.axis_index("c")
    s = lax.axis_index("s")
    wid = s * _NC + c
    b0 = wid * _BPT
    ob0 = (s * _NC + (1 - c)) * _BPT

    pltpu.sync_copy(fixed_hbm, fixed_v)

    def sum_slab(col0, init):
        pltpu.sync_copy(
            idx_hbm.at[pl.ds(0, _NUM_FIELDS), pl.ds(col0, _HSLAB)], slab_v)

        def per_field(f, carry):
            def add8(j, acc):
                for u in range(8):
                    acc = acc + slab_v[f, pl.ds((j * 8 + u) * _L, _L)]
                return acc

            acc0 = jnp.zeros((_L,), jnp.int32) if init else \
                part_v[pl.ds(f * _L, _L)]
            part_v[pl.ds(f * _L, _L)] = lax.fori_loop(
                0, _HSLAB // (8 * _L), add8, acc0)
            return carry

        lax.fori_loop(0, _NUM_FIELDS, per_field, 0)

    sum_slab(ob0, True)
    sum_slab(ob0 + _HSLAB, False)
    sum_slab(b0, False)
    sum_slab(b0 + _HSLAB, False)

    pltpu.sync_copy(part_v, shared.at[s])
    plsc.subcore_barrier()

    def zero_tot(f, carry):
        tot_v[pl.ds(f * _L, _L)] = jnp.zeros((_L,), jnp.int32)
        return carry

    lax.fori_loop(0, _NUM_FIELDS, zero_tot, 0)
    for k in range(_NS // 4):
        pltpu.sync_copy(shared.at[pl.ds(k * 4, 4)], sums4)

        def add_chunk(f, carry):
            t = tot_v[pl.ds(f * _L, _L)]
            for r in range(4):
                t = t + sums4[r, pl.ds(f * _L, _L)]
            tot_v[pl.ds(f * _L, _L)] = t
            return carry

        lax.fori_loop(0, _NUM_FIELDS, add_chunk, 0)

    lanes = lax.iota(jnp.int32, _L)

    def per_row(i, carry):
        r = wid * _RPT + i
        f = lax.div(r, _DIM)
        d = lax.rem(r, _DIM)
        pltpu.sync_copy(table_hbm.at[r], rbuf)
        masked = jnp.sum(tot_v[pl.ds(f * _L, _L)]) == 0

        def quarter(q, carry2):
            col0 = q * _QB
            pltpu.sync_copy(idx_hbm.at[f, pl.ds(col0, _QB)], ibuf)

            @pl.when(jnp.logical_not(masked))
            def _():
                def g8(j, carry3):
                    for u in range(8):
                        off = (j * 8 + u) * _L
                        iv = ibuf[pl.ds(off, _L)]
                        obuf[pl.ds(off, _L)] = plsc.load_gather(rbuf, [iv])
                    return carry3

                lax.fori_loop(0, _QB // (8 * _L), g8, 0)

            @pl.when(masked)
            def _():
                fv = fixed_v[pl.ds(d, _L)][0]
                splat = jnp.full((_L,), fv, jnp.float32)

                def fill(j, carry3):
                    for u in range(8):
                        obuf[pl.ds((j * 8 + u) * _L, _L)] = splat
                    return carry3

                lax.fori_loop(0, _QB // (8 * _L), fill, 0)

            pltpu.sync_copy(obuf, out_hbm.at[r, pl.ds(col0, _QB)])
            return carry2

        lax.fori_loop(0, _NQ, quarter, 0)
        return carry

    lax.fori_loop(0, _RPT, per_row, 0)


@functools.partial(
    pl.kernel,
    out_type=jax.ShapeDtypeStruct((_NROWS, _BATCH), jnp.float32),
    mesh=plsc.VectorSubcoreMesh(core_axis_name="c", subcore_axis_name="s"),
    compiler_params=pltpu.CompilerParams(
        needs_layout_passes=False, use_tc_tiling_on_sc=True),
    scratch_types=[
        pltpu.VMEM((_VOCAB,), jnp.float32),
        pltpu.VMEM((_QB,), jnp.int32),
        pltpu.VMEM((_QB,), jnp.float32),
        pltpu.VMEM((_NUM_FIELDS, _HSLAB), jnp.int32),
        pltpu.VMEM((_NUM_FIELDS * _L,), jnp.int32),
        pltpu.VMEM((_NUM_FIELDS * _L,), jnp.int32),
        pltpu.VMEM((_DIM + _L,), jnp.float32),
        pltpu.VMEM((4, _NUM_FIELDS * _L), jnp.int32),
        pltpu.VMEM_SHARED((_NS, _NUM_FIELDS * _L), jnp.int32),
    ],
)
def _sc_embedding(idx_hbm, table_hbm, fixed_hbm, out_hbm, *scratch):
    _body(idx_hbm, table_hbm, fixed_hbm, out_hbm, *scratch)


def kernel(sparse_inputs, tables, fixed_vector):
    idx_t = sparse_inputs.astype(jnp.int32).T
    table_t = tables.transpose(0, 2, 1).reshape(_NROWS, _VOCAB)
    fixed = jnp.pad(fixed_vector.astype(jnp.float32), (0, _L))
    out2 = _sc_embedding(idx_t, table_t, fixed)
    return out2.reshape(_NUM_FIELDS, _DIM, _BATCH).transpose(2, 0, 1)
